# baseline (device time: 440010 ns/iter reference)
import functools

import jax
import jax.numpy as jnp
from jax import lax
from jax.experimental import pallas as pl
from jax.experimental.pallas import tpu as pltpu

N_DEV = 32
ROWS = 2048
COLS = 2048
CH = ROWS // N_DEV


def _allreduce_body(kv_ref, out_ref, stage_ref, send_sems, recv_sems):
    my = lax.axis_index("i")
    left = (my - 1) % N_DEV
    right = (my + 1) % N_DEV

    barrier_sem = pltpu.get_barrier_semaphore()
    for nbr in (left, right):
        pl.semaphore_signal(
            barrier_sem, inc=1,
            device_id=(nbr,), device_id_type=pl.DeviceIdType.MESH,
        )
    pl.semaphore_wait(barrier_sem, 2)

    out_ref[...] = kv_ref[...]

    for s in range(N_DEV - 1):
        send_idx = (my - s) % N_DEV
        rdma = pltpu.make_async_remote_copy(
            src_ref=out_ref.at[pl.ds(send_idx * CH, CH), :],
            dst_ref=stage_ref.at[s],
            send_sem=send_sems.at[s],
            recv_sem=recv_sems.at[s],
            device_id=(right,),
            device_id_type=pl.DeviceIdType.MESH,
        )
        rdma.start()
        rdma.wait()
        recv_idx = (my - s - 1) % N_DEV
        cur = out_ref[pl.ds(recv_idx * CH, CH), :]
        out_ref[pl.ds(recv_idx * CH, CH), :] = cur + stage_ref[s, :, :]

    for s in range(N_DEV - 1):
        send_idx = (my + 1 - s) % N_DEV
        rdma = pltpu.make_async_remote_copy(
            src_ref=out_ref.at[pl.ds(send_idx * CH, CH), :],
            dst_ref=out_ref.at[pl.ds(send_idx * CH, CH), :],
            send_sem=send_sems.at[N_DEV - 1 + s],
            recv_sem=recv_sems.at[N_DEV - 1 + s],
            device_id=(right,),
            device_id_type=pl.DeviceIdType.MESH,
        )
        rdma.start()
        rdma.wait()

    @functools.partial(pl.run_scoped, sem2=pltpu.SemaphoreType.REGULAR)
    def _(sem2):
        for nbr in (left, right):
            pl.semaphore_signal(
                sem2, inc=1,
                device_id=(nbr,), device_id_type=pl.DeviceIdType.MESH,
            )
        pl.semaphore_wait(sem2, 2)


def _pallas_allreduce(kv):
    return pl.pallas_call(
        _allreduce_body,
        out_shape=jax.ShapeDtypeStruct((ROWS, COLS), jnp.bfloat16),
        in_specs=[pl.BlockSpec(memory_space=pltpu.VMEM)],
        out_specs=pl.BlockSpec(memory_space=pltpu.VMEM),
        scratch_shapes=[
            pltpu.VMEM((N_DEV - 1, CH, COLS), jnp.bfloat16),
            pltpu.SemaphoreType.DMA((2 * (N_DEV - 1),)),
            pltpu.SemaphoreType.DMA((2 * (N_DEV - 1),)),
        ],
        compiler_params=pltpu.CompilerParams(collective_id=0),
    )(kv)


def kernel(x, Wdkv, Wuk, Wuv, Wq, Wqr, Wkr, Wo):
    S, H, Dh, Dr = 1024, 16, 128, 32
    bf = jnp.bfloat16

    xb = x[0].astype(bf)
    c = xb @ Wdkv.astype(bf)
    Kp = c @ Wuk.astype(bf)
    Vp = c @ Wuv.astype(bf)
    kv = jnp.concatenate([Kp, Vp], axis=0)

    kv_sum = _pallas_allreduce(kv)

    K = kv_sum[:S].reshape(S, H, Dh)
    V = kv_sum[S:].reshape(S, H, Dh)
    Q = (xb @ Wq.astype(bf)).reshape(S, H, Dh)
    Qr = (xb @ Wqr.astype(bf)).reshape(S, H, Dr)
    Kr = xb @ Wkr.astype(bf)

    scale = (Dh + Dr) ** -0.5
    scores = jnp.einsum("shd,thd->hst", Q, K,
                        preferred_element_type=jnp.float32)
    scores += jnp.einsum("shd,td->hst", Qr, Kr,
                         preferred_element_type=jnp.float32)
    P = jax.nn.softmax(scores * scale, axis=-1).astype(bf)
    O = jnp.einsum("hst,thd->shd", P, V).reshape(S, H * Dh)
    return (O @ Wo.astype(bf)).astype(jnp.float32)[None]
